# baseline (speedup 1.0000x reference)
import functools

import jax
import jax.numpy as jnp
from jax import lax
from jax.experimental import pallas as pl
from jax.experimental.pallas import tpu as pltpu
from jax.experimental.pallas import tpu_sc as plsc

_VOCAB = 100000
_SYN_VOCAB = 1024
_D = 64
_PAD_D = 16
_LANG_D = 8
_B, _L = 8, 64
_TOK = _B * _L
_SCALE = 8.0

_NC, _NS = 2, 16
_NW = _NC * _NS
_TPW = _TOK // _NW

_TILE = 8192
_GRID = (_VOCAB + _TILE - 1) // _TILE


def _sc_gather_body(t1_hbm, t2_hbm, idx_hbm, o1_hbm, o2_hbm, oe_hbm,
                    idx_v, r1, r2, o1v, o2v, e_v, s1, s2, s3):
    wid = lax.axis_index("s") * _NC + lax.axis_index("c")
    base = wid * _TPW
    pltpu.sync_copy(idx_hbm.at[pl.ds(base, _TPW)], idx_v)
    row_idx = idx_v[...]
    lane = lax.iota(jnp.int32, 16)
    waits = []
    scal = []
    for t in range(_TPW):
        s = jnp.sum(jnp.where(lane == t, row_idx, 0))
        scal.append(s)
        waits.append(pltpu.async_copy(t1_hbm.at[s >> 3], r1.at[t], s1))
        waits.append(pltpu.async_copy(t2_hbm.at[s >> 3], r2.at[t], s2))
    zeros16 = jnp.zeros((16,), jnp.float32)
    for t in range(_TPW):
        for j in range(_SYN_VOCAB // 16):
            e_v[t, pl.ds(j * 16, 16)] = zeros16
    plsc.store_scatter(e_v, [lane, row_idx & (_SYN_VOCAB - 1)],
                       jnp.ones((16,), jnp.float32))
    ce = pltpu.async_copy(e_v, oe_hbm.at[pl.ds(base, _TPW)], s3)
    for w in waits:
        w.wait()
    for t in range(_TPW):
        r = scal[t] & 7
        for j in range(_D // 16):
            sl = pl.ds(j * 16, 16)
            o1v[t, sl] = r1[t, r, sl]
            o2v[t, sl] = r2[t, r, sl]
    pltpu.sync_copy(o1v, o1_hbm.at[pl.ds(base, _TPW)])
    pltpu.sync_copy(o2v, o2_hbm.at[pl.ds(base, _TPW)])
    ce.wait()


@functools.lru_cache(maxsize=1)
def _sc_gather_kernel():
    return pl.kernel(
        _sc_gather_body,
        out_type=[
            jax.ShapeDtypeStruct((_TOK, _D), jnp.float32),
            jax.ShapeDtypeStruct((_TOK, _D), jnp.float32),
            jax.ShapeDtypeStruct((_TOK, _SYN_VOCAB), jnp.float32),
        ],
        mesh=plsc.VectorSubcoreMesh(core_axis_name="c", subcore_axis_name="s"),
        scratch_types=[
            pltpu.VMEM((_TPW,), jnp.int32),
            pltpu.VMEM((_TPW, 8, _D), jnp.float32),
            pltpu.VMEM((_TPW, 8, _D), jnp.float32),
            pltpu.VMEM((_TPW, _D), jnp.float32),
            pltpu.VMEM((_TPW, _D), jnp.float32),
            pltpu.VMEM((_TPW, _SYN_VOCAB), jnp.float32),
            pltpu.SemaphoreType.DMA,
            pltpu.SemaphoreType.DMA,
            pltpu.SemaphoreType.DMA,
        ],
        compiler_params=pltpu.CompilerParams(needs_layout_passes=False),
    )


def _sc_gather(t1, t2, idx):
    t1_3d = t1.reshape(_VOCAB // 8, 8, _D)
    t2_3d = t2.reshape(_VOCAB // 8, 8, _D)
    return _sc_gather_kernel()(t1_3d, t2_3d, idx)


def _three_stage(x):
    x = x * _SCALE
    steep = 3.0
    pos = jax.nn.sigmoid(steep * (x - 4.0))
    neg = jax.nn.sigmoid(steep * (-x - 4.0))
    return pos - neg


def _tc_body(e1_ref, e2_ref, wse_ref, bse_ref, wss_ref, bss_ref,
             wrev_ref, brev_ref,
             big_ref, sym_ref, syn_ref, ssum_ref, sp_ref, syn_sc_ref):
    i = pl.program_id(0)

    @pl.when(i == 0)
    def _critical():
        syn = _three_stage(e2_ref[...] * (1.0 / _SCALE))
        sfs = jnp.dot(syn, wse_ref[...],
                      preferred_element_type=jnp.float32) + bse_ref[...]
        sp_ref[...] = e1_ref[...] + sfs
        syn_sc_ref[...] = syn

    @pl.when(i == 1)
    def _small():
        syn = syn_sc_ref[...]
        sym_ref[...] = sp_ref[...]
        syn_ref[...] = syn
        ssum_ref[...] = jnp.dot(syn, wss_ref[...],
                                preferred_element_type=jnp.float32) + bss_ref[...]

    wr = wrev_ref[...]
    bias = 0.1 * jnp.sum(wr[_D:, :], axis=0, keepdims=True) + brev_ref[...]
    big_ref[...] = (
        jnp.dot(sp_ref[...].astype(jnp.bfloat16),
                wr[:_D, :].astype(jnp.bfloat16),
                preferred_element_type=jnp.float32)
        + bias)


def _tc_dense(emb1, emb2, wse, bse, wss, bss, wrev, brev):
    return pl.pallas_call(
        _tc_body,
        grid=(_GRID,),
        in_specs=[
            pl.BlockSpec((_TOK, _D), lambda i: (0, 0)),
            pl.BlockSpec((_TOK, _D), lambda i: (0, 0)),
            pl.BlockSpec((_D, _D), lambda i: (0, 0)),
            pl.BlockSpec((1, _D), lambda i: (0, 0)),
            pl.BlockSpec((_D, _SYN_VOCAB), lambda i: (0, 0)),
            pl.BlockSpec((1, _SYN_VOCAB), lambda i: (0, 0)),
            pl.BlockSpec((_D + _PAD_D, _TILE), lambda i: (0, i)),
            pl.BlockSpec((1, _TILE), lambda i: (0, i)),
        ],
        out_specs=[
            pl.BlockSpec((_TOK, _TILE), lambda i: (0, i)),
            pl.BlockSpec((_TOK, _D), lambda i: (0, 0)),
            pl.BlockSpec((_TOK, _D), lambda i: (0, 0)),
            pl.BlockSpec((_TOK, _SYN_VOCAB), lambda i: (0, 0)),
        ],
        out_shape=[
            jax.ShapeDtypeStruct((_TOK, _VOCAB), jnp.float32),
            jax.ShapeDtypeStruct((_TOK, _D), jnp.float32),
            jax.ShapeDtypeStruct((_TOK, _D), jnp.float32),
            jax.ShapeDtypeStruct((_TOK, _SYN_VOCAB), jnp.float32),
        ],
        scratch_shapes=[pltpu.VMEM((_TOK, _D), jnp.float32),
                        pltpu.VMEM((_TOK, _D), jnp.float32)],
        compiler_params=pltpu.CompilerParams(
            vmem_limit_bytes=100 * 1024 * 1024),
    )(emb1, emb2, wse, bse, wss, bss, wrev, brev)


def kernel(ids, table_v1, table_v2, W_syn_emb, b_syn_emb, W_syn_sum,
           b_syn_sum, W_rev, b_rev):
    ids_flat = ids.reshape(_TOK).astype(jnp.int32)
    emb1, emb2, exp = _sc_gather(table_v1, table_v2, ids_flat)
    big, symbol, syn, ssum = _tc_dense(
        emb1, emb2,
        W_syn_emb, b_syn_emb.reshape(1, _D),
        W_syn_sum, b_syn_sum.reshape(1, _SYN_VOCAB),
        W_rev, b_rev.reshape(1, _VOCAB))
    pad_lang = jnp.full((_B, _L, _PAD_D + _LANG_D), 0.1, dtype=jnp.float32)
    full = jnp.concatenate(
        [symbol.reshape(_B, _L, _D), pad_lang, syn.reshape(_B, _L, _D)],
        axis=2)
    return (full,
            big.reshape(_B, _L, _VOCAB),
            ssum.reshape(_B, _L, _SYN_VOCAB),
            exp.reshape(_B, _L, _SYN_VOCAB))


# speedup vs baseline: 1.0144x; 1.0144x over previous
import functools

import jax
import jax.numpy as jnp
from jax import lax
from jax.experimental import pallas as pl
from jax.experimental.pallas import tpu as pltpu
from jax.experimental.pallas import tpu_sc as plsc

_VOCAB = 100000
_SYN_VOCAB = 1024
_D = 64
_PAD_D = 16
_LANG_D = 8
_B, _L = 8, 64
_TOK = _B * _L
_SCALE = 8.0

_NC, _NS = 2, 16
_NW = _NC * _NS
_TPW = _TOK // _NW

_TILE = 8192
_GRID = (_VOCAB + _TILE - 1) // _TILE


def _sc_gather_body(t1_hbm, t2_hbm, idx_hbm, o1_hbm, o2_hbm,
                    idx_v, r1, r2, o1v, o2v, s1, s2):
    wid = lax.axis_index("s") * _NC + lax.axis_index("c")
    base = wid * _TPW
    pltpu.sync_copy(idx_hbm.at[pl.ds(base, _TPW)], idx_v)
    row_idx = idx_v[...]
    lane = lax.iota(jnp.int32, 16)
    waits = []
    scal = []
    for t in range(_TPW):
        s = jnp.sum(jnp.where(lane == t, row_idx, 0))
        scal.append(s)
        waits.append(pltpu.async_copy(t1_hbm.at[s >> 3], r1.at[t], s1))
        waits.append(pltpu.async_copy(t2_hbm.at[s >> 3], r2.at[t], s2))
    for w in waits:
        w.wait()
    for t in range(_TPW):
        r = scal[t] & 7
        for j in range(_D // 16):
            sl = pl.ds(j * 16, 16)
            o1v[t, sl] = r1[t, r, sl]
            o2v[t, sl] = r2[t, r, sl]
    pltpu.sync_copy(o1v, o1_hbm.at[pl.ds(base, _TPW)])
    pltpu.sync_copy(o2v, o2_hbm.at[pl.ds(base, _TPW)])


@functools.lru_cache(maxsize=1)
def _sc_gather_kernel():
    return pl.kernel(
        _sc_gather_body,
        out_type=[
            jax.ShapeDtypeStruct((_TOK, _D), jnp.float32),
            jax.ShapeDtypeStruct((_TOK, _D), jnp.float32),
        ],
        mesh=plsc.VectorSubcoreMesh(core_axis_name="c", subcore_axis_name="s"),
        scratch_types=[
            pltpu.VMEM((_TPW,), jnp.int32),
            pltpu.VMEM((_TPW, 8, _D), jnp.float32),
            pltpu.VMEM((_TPW, 8, _D), jnp.float32),
            pltpu.VMEM((_TPW, _D), jnp.float32),
            pltpu.VMEM((_TPW, _D), jnp.float32),
            pltpu.SemaphoreType.DMA,
            pltpu.SemaphoreType.DMA,
        ],
        compiler_params=pltpu.CompilerParams(needs_layout_passes=False),
    )


def _sc_gather(t1, t2, idx):
    t1_3d = t1.reshape(_VOCAB // 8, 8, _D)
    t2_3d = t2.reshape(_VOCAB // 8, 8, _D)
    return _sc_gather_kernel()(t1_3d, t2_3d, idx)


def _three_stage(x):
    x = x * _SCALE
    steep = 3.0
    pos = jax.nn.sigmoid(steep * (x - 4.0))
    neg = jax.nn.sigmoid(steep * (-x - 4.0))
    return pos - neg


def _tc_body(ids_ref, e1_ref, e2_ref, wse_ref, bse_ref, wss_ref, bss_ref,
             wrev_ref, brev_ref,
             big_ref, sym_ref, syn_ref, ssum_ref, exp_ref, sp_ref,
             syn_sc_ref):
    i = pl.program_id(0)

    @pl.when(i == 0)
    def _critical():
        syn = _three_stage(e2_ref[...] * (1.0 / _SCALE))
        sfs = jnp.dot(syn, wse_ref[...],
                      preferred_element_type=jnp.float32) + bse_ref[...]
        sp_ref[...] = e1_ref[...] + sfs
        syn_sc_ref[...] = syn

    @pl.when(i == 1)
    def _small():
        syn = syn_sc_ref[...]
        sym_ref[...] = sp_ref[...]
        syn_ref[...] = syn
        ssum_ref[...] = jnp.dot(syn, wss_ref[...],
                                preferred_element_type=jnp.float32) + bss_ref[...]
        mod = lax.rem(ids_ref[...], _SYN_VOCAB)
        cols = lax.broadcasted_iota(jnp.int32, (_TOK, _SYN_VOCAB), 1)
        exp_ref[...] = (cols == mod).astype(jnp.float32)

    wr = wrev_ref[...]
    bias = 0.1 * jnp.sum(wr[_D:, :], axis=0, keepdims=True) + brev_ref[...]
    big_ref[...] = (
        jnp.dot(sp_ref[...].astype(jnp.bfloat16),
                wr[:_D, :].astype(jnp.bfloat16),
                preferred_element_type=jnp.float32)
        + bias)


def _tc_dense(ids_col, emb1, emb2, wse, bse, wss, bss, wrev, brev):
    return pl.pallas_call(
        _tc_body,
        grid=(_GRID,),
        in_specs=[
            pl.BlockSpec((_TOK, 1), lambda i: (0, 0)),
            pl.BlockSpec((_TOK, _D), lambda i: (0, 0)),
            pl.BlockSpec((_TOK, _D), lambda i: (0, 0)),
            pl.BlockSpec((_D, _D), lambda i: (0, 0)),
            pl.BlockSpec((1, _D), lambda i: (0, 0)),
            pl.BlockSpec((_D, _SYN_VOCAB), lambda i: (0, 0)),
            pl.BlockSpec((1, _SYN_VOCAB), lambda i: (0, 0)),
            pl.BlockSpec((_D + _PAD_D, _TILE), lambda i: (0, i)),
            pl.BlockSpec((1, _TILE), lambda i: (0, i)),
        ],
        out_specs=[
            pl.BlockSpec((_TOK, _TILE), lambda i: (0, i)),
            pl.BlockSpec((_TOK, _D), lambda i: (0, 0)),
            pl.BlockSpec((_TOK, _D), lambda i: (0, 0)),
            pl.BlockSpec((_TOK, _SYN_VOCAB), lambda i: (0, 0)),
            pl.BlockSpec((_TOK, _SYN_VOCAB), lambda i: (0, 0)),
        ],
        out_shape=[
            jax.ShapeDtypeStruct((_TOK, _VOCAB), jnp.float32),
            jax.ShapeDtypeStruct((_TOK, _D), jnp.float32),
            jax.ShapeDtypeStruct((_TOK, _D), jnp.float32),
            jax.ShapeDtypeStruct((_TOK, _SYN_VOCAB), jnp.float32),
            jax.ShapeDtypeStruct((_TOK, _SYN_VOCAB), jnp.float32),
        ],
        scratch_shapes=[pltpu.VMEM((_TOK, _D), jnp.float32),
                        pltpu.VMEM((_TOK, _D), jnp.float32)],
        compiler_params=pltpu.CompilerParams(
            vmem_limit_bytes=100 * 1024 * 1024),
    )(ids_col, emb1, emb2, wse, bse, wss, bss, wrev, brev)


def kernel(ids, table_v1, table_v2, W_syn_emb, b_syn_emb, W_syn_sum,
           b_syn_sum, W_rev, b_rev):
    ids_flat = ids.reshape(_TOK).astype(jnp.int32)
    emb1, emb2 = _sc_gather(table_v1, table_v2, ids_flat)
    big, symbol, syn, ssum, exp = _tc_dense(
        ids_flat.reshape(_TOK, 1), emb1, emb2,
        W_syn_emb, b_syn_emb.reshape(1, _D),
        W_syn_sum, b_syn_sum.reshape(1, _SYN_VOCAB),
        W_rev, b_rev.reshape(1, _VOCAB))
    pad_lang = jnp.full((_B, _L, _PAD_D + _LANG_D), 0.1, dtype=jnp.float32)
    full = jnp.concatenate(
        [symbol.reshape(_B, _L, _D), pad_lang, syn.reshape(_B, _L, _D)],
        axis=2)
    return (full,
            big.reshape(_B, _L, _VOCAB),
            ssum.reshape(_B, _L, _SYN_VOCAB),
            exp.reshape(_B, _L, _SYN_VOCAB))
